# Initial kernel scaffold; baseline (speedup 1.0000x reference)
import jax
import jax.numpy as jnp
from jax.experimental import pallas as pl


def kernel(X, edge_index, D, n1_pW, n1_pb, n1_sW, n1_sb, e1_pW, e1_pb, e1_sW, e1_sb, n2_pW, n2_pb, n2_sW, n2_sb, e2_pW, e2_pb, e2_sW, e2_sb, n3_pW, n3_pb, n3_sW, n3_sb):
    raise NotImplementedError("write your pallas kernel here")


# speedup vs baseline: 2.6239x; 2.6239x over previous
import functools

import jax
import jax.numpy as jnp
from jax import lax
from jax.experimental import pallas as pl
from jax.experimental.pallas import tpu as pltpu
from jax.experimental.pallas import tpu_sc as plsc

N = 10000
E = 320000
DF = 128
H = 32
H2 = 2 * H
K = 30

NC = 2
NS = 16
NW = NC * NS
EPT = E // NW
CH = 80
NCHUNK = EPT // CH
N2 = 10240
RPT = N2 // NS
RCH = 64
NRCH = RPT // RCH

_MESH = plsc.VectorSubcoreMesh(core_axis_name="c", subcore_axis_name="s")
_UNTILED = pltpu.CompilerParams(use_tc_tiling_on_sc=False)


def _zero_stage(stage, width):
    zero16 = jnp.zeros((16,), jnp.float32)

    def zrow(i, _):
        for j in range(width // 16):
            stage[i, pl.ds(16 * j, 16)] = zero16
        return 0

    lax.fori_loop(0, RCH, zrow, 0)


def _phase_a(row1, col1, x):
    @functools.partial(
        pl.kernel,
        out_type=[
            jax.ShapeDtypeStruct((E, DF), jnp.float32),
            jax.ShapeDtypeStruct((2 * N2, DF), jnp.float32),
        ],
        mesh=_MESH,
        scratch_types=[
            pltpu.VMEM((CH,), jnp.int32),
            pltpu.VMEM((CH,), jnp.int32),
            pltpu.VMEM((CH, DF), jnp.float32),
            pltpu.VMEM((CH, DF), jnp.float32),
            pltpu.VMEM((CH, DF), jnp.float32),
            pltpu.VMEM((RCH, DF), jnp.float32),
            pltpu.VMEM_SHARED((N2, DF), jnp.float32),
            pltpu.SemaphoreType.DMA,
            pltpu.SemaphoreType.DMA,
        ],
    )
    def body(row_hbm, col_hbm, x_hbm, av_out, asum_out,
             idx_r, idx_c, xr, xc, av, stage, acc, sem1, sem2):
        cid = lax.axis_index("c")
        sid = lax.axis_index("s")
        wid = cid * NS + sid

        _zero_stage(stage, DF)
        for j in range(NRCH):
            pltpu.sync_copy(stage, acc.at[pl.ds(sid * RPT + j * RCH, RCH)])
        plsc.subcore_barrier()

        def chunk(c, _):
            base = wid * EPT + c * CH
            pltpu.sync_copy(row_hbm.at[pl.ds(base, CH)], idx_r)
            pltpu.sync_copy(col_hbm.at[pl.ds(base, CH)], idx_c)
            cp1 = pltpu.async_copy(x_hbm.at[idx_r], xr, sem1)
            cp2 = pltpu.async_copy(x_hbm.at[idx_c], xc, sem2)
            cp1.wait()
            cp2.wait()

            def rowbody(i, _):
                for j in range(DF // 16):
                    s = 16 * j
                    av[i, pl.ds(s, 16)] = jnp.abs(
                        xr[i, pl.ds(s, 16)] - xc[i, pl.ds(s, 16)])
                return 0

            lax.fori_loop(0, CH, rowbody, 0)
            pltpu.sync_copy(av, av_out.at[pl.ds(base, CH)])
            pltpu.sync_copy(av, acc.at[idx_r], add=True)
            return 0

        lax.fori_loop(0, NCHUNK, chunk, 0)

        plsc.subcore_barrier()
        for j in range(NRCH):
            off = sid * RPT + j * RCH
            pltpu.sync_copy(acc.at[pl.ds(off, RCH)], stage)
            pltpu.sync_copy(stage, asum_out.at[pl.ds(cid * N2 + off, RCH)])

    return body(row1, col1, x)


def _gather_pair(row1, col1, xn):
    @functools.partial(
        pl.kernel,
        out_type=jax.ShapeDtypeStruct((E, H2), jnp.float32),
        mesh=_MESH,
        scratch_types=[
            pltpu.VMEM((CH,), jnp.int32),
            pltpu.VMEM((CH,), jnp.int32),
            pltpu.VMEM((CH, H), jnp.float32),
            pltpu.VMEM((CH, H), jnp.float32),
            pltpu.VMEM((CH, H2), jnp.float32),
            pltpu.SemaphoreType.DMA,
            pltpu.SemaphoreType.DMA,
        ],
        compiler_params=_UNTILED,
    )
    def body(row_hbm, col_hbm, xn_hbm, ec_out, idx_r, idx_c, xr, xc, ec,
             sem1, sem2):
        cid = lax.axis_index("c")
        sid = lax.axis_index("s")
        wid = cid * NS + sid

        def chunk(c, _):
            base = wid * EPT + c * CH
            pltpu.sync_copy(row_hbm.at[pl.ds(base, CH)], idx_r)
            pltpu.sync_copy(col_hbm.at[pl.ds(base, CH)], idx_c)
            cp1 = pltpu.async_copy(xn_hbm.at[idx_r], xr, sem1)
            cp2 = pltpu.async_copy(xn_hbm.at[idx_c], xc, sem2)
            cp1.wait()
            cp2.wait()

            def rowbody(i, _):
                for j in range(H // 16):
                    s = 16 * j
                    a = xr[i, pl.ds(s, 16)]
                    b = xc[i, pl.ds(s, 16)]
                    ec[i, pl.ds(s, 16)] = (a - b) * 0.5
                    ec[i, pl.ds(H + s, 16)] = (a + b) * 0.5
                return 0

            lax.fori_loop(0, CH, rowbody, 0)
            pltpu.sync_copy(ec, ec_out.at[pl.ds(base, CH)])
            return 0

        lax.fori_loop(0, NCHUNK, chunk, 0)

    return body(row1, col1, xn)


def _scatter_sum(row1, vals):
    @functools.partial(
        pl.kernel,
        out_type=jax.ShapeDtypeStruct((2 * N2, H), jnp.float32),
        mesh=_MESH,
        scratch_types=[
            pltpu.VMEM((CH,), jnp.int32),
            pltpu.VMEM((CH, H), jnp.float32),
            pltpu.VMEM((RCH, H), jnp.float32),
            pltpu.VMEM_SHARED((N2, H), jnp.float32),
            pltpu.SemaphoreType.DMA,
        ],
        compiler_params=_UNTILED,
    )
    def body(row_hbm, v_hbm, acc_out, idx_r, vb, stage, acc, sem1):
        cid = lax.axis_index("c")
        sid = lax.axis_index("s")
        wid = cid * NS + sid

        _zero_stage(stage, H)
        for j in range(NRCH):
            pltpu.sync_copy(stage, acc.at[pl.ds(sid * RPT + j * RCH, RCH)])
        plsc.subcore_barrier()

        def chunk(c, _):
            base = wid * EPT + c * CH
            pltpu.sync_copy(row_hbm.at[pl.ds(base, CH)], idx_r)
            pltpu.async_copy(v_hbm.at[pl.ds(base, CH)], vb, sem1).wait()
            pltpu.sync_copy(vb, acc.at[idx_r], add=True)
            return 0

        lax.fori_loop(0, NCHUNK, chunk, 0)

        plsc.subcore_barrier()
        for j in range(NRCH):
            off = sid * RPT + j * RCH
            pltpu.sync_copy(acc.at[pl.ds(off, RCH)], stage)
            pltpu.sync_copy(stage, acc_out.at[pl.ds(cid * N2 + off, RCH)])

    return body(row1, vals)


def _dotT(a, b):
    return lax.dot_general(a, b, (((1,), (1,)), ((), ())),
                           preferred_element_type=jnp.float32)


_BE = 2560


def _ew_matmul(a, w, b):
    din = a.shape[1]

    def body(a_ref, w_ref, b_ref, o_ref):
        o_ref[...] = _dotT(a_ref[...], w_ref[...]) + b_ref[...]

    return pl.pallas_call(
        body,
        grid=(E // _BE,),
        in_specs=[
            pl.BlockSpec((_BE, din), lambda i: (i, 0)),
            pl.BlockSpec((H, din), lambda i: (0, 0)),
            pl.BlockSpec((1, H), lambda i: (0, 0)),
        ],
        out_specs=pl.BlockSpec((_BE, H), lambda i: (i, 0)),
        out_shape=jax.ShapeDtypeStruct((E, H), jnp.float32),
    )(a, w, b)


def _edge_conv1(ecat, pw, pb, s1, sw2, sb2):
    def body(ec, pwr, pbr, s1r, sw2r, sb2r, av_o, s2_o):
        av = jnp.maximum((_dotT(ec[...], pwr[...]) + pbr[...]) + s1r[...], 0.0)
        av_o[...] = av
        s2_o[...] = _dotT(av, sw2r[...]) + sb2r[...]

    o = jax.ShapeDtypeStruct((E, H), jnp.float32)
    espec = pl.BlockSpec((_BE, H), lambda i: (i, 0))
    return pl.pallas_call(
        body,
        grid=(E // _BE,),
        in_specs=[
            pl.BlockSpec((_BE, H2), lambda i: (i, 0)),
            pl.BlockSpec((H, H2), lambda i: (0, 0)),
            pl.BlockSpec((1, H), lambda i: (0, 0)),
            espec,
            pl.BlockSpec((H, H), lambda i: (0, 0)),
            pl.BlockSpec((1, H), lambda i: (0, 0)),
        ],
        out_specs=[espec, espec],
        out_shape=[o, o],
    )(ecat, pw, pb, s1, sw2, sb2)


def _edge_conv2(ecat, pw, pb, s2):
    def body(ec, pwr, pbr, s2r, av_o):
        av_o[...] = jnp.maximum(
            (_dotT(ec[...], pwr[...]) + pbr[...]) + s2r[...], 0.0)

    espec = pl.BlockSpec((_BE, H), lambda i: (i, 0))
    return pl.pallas_call(
        body,
        grid=(E // _BE,),
        in_specs=[
            pl.BlockSpec((_BE, H2), lambda i: (i, 0)),
            pl.BlockSpec((H, H2), lambda i: (0, 0)),
            pl.BlockSpec((1, H), lambda i: (0, 0)),
            espec,
        ],
        out_specs=espec,
        out_shape=jax.ShapeDtypeStruct((E, H), jnp.float32),
    )(ecat, pw, pb, s2)


_BN = 2000


def _node1(a0, a1, dcol, w1, pb, sb):
    def body(a0r, a1r, d, w1r, pbr, sbr, xn_o):
        xs = (a0r[...] + a1r[...]) / d[...]
        xn_o[...] = jnp.maximum((_dotT(xs, w1r[...]) + pbr[...]) + sbr[...],
                                0.0)

    din = a0.shape[-1]
    aspec = pl.BlockSpec((_BN, din), lambda i: (i, 0))
    bspec = pl.BlockSpec((1, H), lambda i: (0, 0))
    return pl.pallas_call(
        body,
        grid=(N // _BN,),
        in_specs=[
            aspec,
            aspec,
            pl.BlockSpec((_BN, 1), lambda i: (i, 0)),
            pl.BlockSpec((H, din), lambda i: (0, 0)),
            bspec,
            bspec,
        ],
        out_specs=pl.BlockSpec((_BN, H), lambda i: (i, 0)),
        out_shape=jax.ShapeDtypeStruct((N, H), jnp.float32),
    )(a0, a1, dcol, w1, pb, sb)


def _node2(a0, a1, dcol, xn1, wp, pb, ws, sb):
    def body(a0r, a1r, d, x1, wpr, pbr, wsr, sbr, xn_o):
        xs = (a0r[...] + a1r[...]) / d[...]
        xn_o[...] = jnp.maximum(
            (_dotT(xs, wpr[...]) + pbr[...])
            + (_dotT(x1[...], wsr[...]) + sbr[...]), 0.0)

    din = a0.shape[-1]
    aspec = pl.BlockSpec((_BN, din), lambda i: (i, 0))
    bspec = pl.BlockSpec((1, H), lambda i: (0, 0))
    sspec = pl.BlockSpec((H, H), lambda i: (0, 0))
    nspec = pl.BlockSpec((_BN, H), lambda i: (i, 0))
    return pl.pallas_call(
        body,
        grid=(N // _BN,),
        in_specs=[
            aspec,
            aspec,
            pl.BlockSpec((_BN, 1), lambda i: (i, 0)),
            nspec,
            sspec,
            bspec,
            sspec,
            bspec,
        ],
        out_specs=nspec,
        out_shape=jax.ShapeDtypeStruct((N, H), jnp.float32),
    )(a0, a1, dcol, xn1, wp, pb, ws, sb)


def _node3(a0, a1, dcol, xn2, w3p, pb3, w3s, sb3):
    def body(a0r, a1r, d, x2, wpr, pbr, wsr, sbr, fx_o):
        xs = (a0r[...] + a1r[...]) / d[...]
        pre = (_dotT(xs, wpr[...]) + pbr[...]) + (_dotT(x2[...], wsr[...])
                                                  + sbr[...])
        m = jnp.max(pre, axis=-1, keepdims=True)
        e = jnp.exp(pre - m)
        fx_o[...] = e / jnp.sum(e, axis=-1, keepdims=True)

    din = a0.shape[-1]
    aspec = pl.BlockSpec((_BN, din), lambda i: (i, 0))
    bspec = pl.BlockSpec((1, H), lambda i: (0, 0))
    sspec = pl.BlockSpec((H, H), lambda i: (0, 0))
    nspec = pl.BlockSpec((_BN, H), lambda i: (i, 0))
    return pl.pallas_call(
        body,
        grid=(N // _BN,),
        in_specs=[
            aspec,
            aspec,
            pl.BlockSpec((_BN, 1), lambda i: (i, 0)),
            nspec,
            sspec,
            bspec,
            sspec,
            bspec,
        ],
        out_specs=nspec,
        out_shape=jax.ShapeDtypeStruct((N, H), jnp.float32),
    )(a0, a1, dcol, xn2, w3p, pb3, w3s, sb3)


def _corr(fx32):
    bm = 1024
    gd = pl.cdiv(N, bm)

    def body(fi, fj, o_ref):
        o_ref[...] = 1.0 - _dotT(fi[...], fj[...])

    return pl.pallas_call(
        body,
        grid=(gd, gd),
        in_specs=[
            pl.BlockSpec((bm, H), lambda i, j: (i, 0)),
            pl.BlockSpec((bm, H), lambda i, j: (j, 0)),
        ],
        out_specs=pl.BlockSpec((bm, bm), lambda i, j: (i, j)),
        out_shape=jax.ShapeDtypeStruct((N, N), jnp.float32),
    )(fx32, fx32)


def _split(asum, din):
    a = asum.reshape(2, N2, din)
    return a[0, :N], a[1, :N]


def kernel(X, edge_index, D, n1_pW, n1_pb, n1_sW, n1_sb, e1_pW, e1_pb, e1_sW,
           e1_sb, n2_pW, n2_pb, n2_sW, n2_sb, e2_pW, e2_pb, e2_sW, e2_sb,
           n3_pW, n3_pb, n3_sW, n3_sb):
    f32 = jnp.float32
    row = edge_index[0]
    col = edge_index[1]
    dcol = D.reshape(N, 1)

    av, asum1 = _phase_a(row, col, X)
    a0, a1 = _split(asum1, DF)

    s1 = _ew_matmul(av, e1_sW, e1_sb.reshape(1, H))
    xn1 = _node1(a0, a1, dcol, n1_pW, n1_pb.reshape(1, H),
                 n1_sb.reshape(1, H))

    ecat1 = _gather_pair(row, col, xn1)
    av1, s2 = _edge_conv1(ecat1, e1_pW, e1_pb.reshape(1, H), s1, e2_sW,
                          e2_sb.reshape(1, H))

    asum2 = _scatter_sum(row, av1)
    b0, b1 = _split(asum2, H)
    xn2 = _node2(b0, b1, dcol, xn1, n2_pW, n2_pb.reshape(1, H), n2_sW,
                 n2_sb.reshape(1, H))

    ecat2 = _gather_pair(row, col, xn2)
    av2 = _edge_conv2(ecat2, e2_pW, e2_pb.reshape(1, H), s2)

    asum3 = _scatter_sum(row, av2)
    c0, c1 = _split(asum3, H)

    w3p = jnp.concatenate([n3_pW, jnp.zeros((H - K, H), f32)], 0)
    w3s = jnp.concatenate([n3_sW, jnp.zeros((H - K, H), f32)], 0)
    pb3 = jnp.concatenate([n3_pb, jnp.full((H - K,), -1e30, f32)], 0)
    sb3 = jnp.concatenate([n3_sb, jnp.zeros((H - K,), f32)], 0)
    fx32 = _node3(c0, c1, dcol, xn2, w3p, pb3.reshape(1, H),
                  w3s, sb3.reshape(1, H))

    corr = _corr(fx32)
    return fx32[:, :K], corr
